# baseline (speedup 1.0000x reference)
import functools

import jax
import jax.numpy as jnp
from jax import lax
from jax.experimental import pallas as pl
from jax.experimental.pallas import tpu as pltpu
from jax.experimental.pallas import tpu_sc as plsc

B = 16384
NK = 128
NE = NK - 1
H = 64
BT = 1024
NCHUNK = 4
BC = B // NCHUNK

NW = 32
CH = 128


def _sc_gather(uid, iid, item_diff_w, pri_flat, disc_flat):
    bc = uid.shape[0]
    bpw = bc // NW
    nch = bpw // CH
    mesh = plsc.VectorSubcoreMesh(core_axis_name="c", subcore_axis_name="s")
    f32 = jnp.float32
    out_type = (
        jax.ShapeDtypeStruct((bc, NK), f32),
        jax.ShapeDtypeStruct((bc,), f32),
        jax.ShapeDtypeStruct((bc,), f32),
    )
    scratch = [
        pltpu.VMEM((nch, CH), jnp.int32),
        pltpu.VMEM((nch, CH), jnp.int32),
        pltpu.VMEM((nch, CH), jnp.int32),
        pltpu.VMEM((2, CH, NK), f32),
        pltpu.VMEM((2, CH), f32),
        pltpu.VMEM((2, CH), f32),
    ] + [pltpu.SemaphoreType.DMA] * 4

    @functools.partial(pl.kernel, out_type=out_type, mesh=mesh,
                       scratch_types=scratch)
    def k(uid_h, iid_h, idf_h, pri_h, dsc_h,
          idf_o, pri_o, dsc_o,
          idxu, idxi, pidx, idfb, prib, dscb,
          sg0, sg1, sw0, sw1):
        wid = lax.axis_index("s") * 2 + lax.axis_index("c")
        base = wid * bpw
        for c in range(nch):
            pltpu.sync_copy(uid_h.at[pl.ds(base + c * CH, CH)], idxu.at[c])
            pltpu.sync_copy(iid_h.at[pl.ds(base + c * CH, CH)], idxi.at[c])
        for c in range(nch):
            for j in range(CH // 16):
                sl = pl.ds(j * 16, 16)
                pidx[c, sl] = idxu[c, sl] * NK
        gsem = (sg0, sg1)
        wsem = (sw0, sw1)
        gh = [None, None]
        wh = [None, None]

        def fire_gather(c):
            b = c % 2
            s = gsem[b]
            gh[b] = [
                pltpu.async_copy(idf_h.at[idxi.at[c]], idfb.at[b], s),
                pltpu.async_copy(pri_h.at[pidx.at[c]], prib.at[b], s),
                pltpu.async_copy(dsc_h.at[idxi.at[c]], dscb.at[b], s),
            ]

        def fire_writeback(c):
            b = c % 2
            rb = base + c * CH
            s = wsem[b]
            wh[b] = [
                pltpu.async_copy(idfb.at[b], idf_o.at[pl.ds(rb, CH)], s),
                pltpu.async_copy(prib.at[b], pri_o.at[pl.ds(rb, CH)], s),
                pltpu.async_copy(dscb.at[b], dsc_o.at[pl.ds(rb, CH)], s),
            ]

        def drain(handles):
            for h in handles:
                h.wait()

        fire_gather(0)
        for c in range(nch):
            b = c % 2
            if c + 1 < nch:
                if wh[1 - b] is not None:
                    drain(wh[1 - b])
                fire_gather(c + 1)
            drain(gh[b])
            fire_writeback(c)
        for w in wh:
            if w is not None:
                drain(w)

    return k(uid, iid, item_diff_w, pri_flat, disc_flat)


def _tc_body(pri0_ref, cp_ref, cn_ref, idiff_ref, idisc_ref, know_ref,
             uw_ref, ub_ref, iw_ref, ib_ref, c1w_ref, c1b_ref, c2w_ref,
             c2b_ref, out_ref):
    sig = jax.nn.sigmoid
    bt = pri0_ref.shape[0]
    cp = sig(cp_ref[...])
    cn = sig(cn_ref[...])
    pri0 = sig(pri0_ref[...])
    a = jnp.concatenate([jnp.zeros((bt, 1), jnp.float32), cp - cn], axis=1)
    b = jnp.concatenate([pri0, cn], axis=1)
    d = 1
    for _ in range(7):
        a_sh = jnp.concatenate(
            [jnp.ones((bt, d), jnp.float32), a[:, :NK - d]], axis=1)
        b_sh = jnp.concatenate(
            [jnp.zeros((bt, d), jnp.float32), b[:, :NK - d]], axis=1)
        b = a * b_sh + b
        a = a * a_sh
        d *= 2
    mastery = b

    know = know_ref[...]
    dn = (((1,), (1,)), ((), ()))
    uf = jnp.tanh(
        lax.dot_general(mastery * know, uw_ref[...], dn,
                        preferred_element_type=jnp.float32) + ub_ref[...])
    itf = sig(
        lax.dot_general(sig(idiff_ref[...]) * know, iw_ref[...], dn,
                        preferred_element_type=jnp.float32) + ib_ref[...])
    inp = (uf - itf) * sig(idisc_ref[...])
    x1 = sig(
        lax.dot_general(inp, c1w_ref[...], dn,
                        preferred_element_type=jnp.float32) + c1b_ref[...])
    out_ref[...] = sig(
        jnp.sum(x1 * c2w_ref[...], axis=1, keepdims=True) + c2b_ref[...])


def _tc_forward(chunk, pri0, cp, cn, idiff, idisc, know_full,
                uw, ub, iw, ib, c1w, c1b, c2w, c2b):
    bc = pri0.shape[0]
    nb0 = chunk * (bc // BT)
    row = lambda shape: pl.BlockSpec(shape, lambda i: (i, 0))
    krow = pl.BlockSpec((BT, NK), lambda i: (nb0 + i, 0))
    full = lambda shape: pl.BlockSpec(shape, lambda i: (0, 0))
    in_specs = [
        row((BT, 1)),
        row((BT, NE)),
        row((BT, NE)),
        row((BT, NK)),
        row((BT, 1)),
        krow,
        full((H, NK)),
        full((1, H)),
        full((H, NK)),
        full((1, H)),
        full((H // 2, H)),
        full((1, H // 2)),
        full((1, H // 2)),
        full((1, 1)),
    ]
    return pl.pallas_call(
        _tc_body,
        grid=(bc // BT,),
        in_specs=in_specs,
        out_specs=row((BT, 1)),
        out_shape=jax.ShapeDtypeStruct((bc, 1), jnp.float32),
    )(pri0, cp, cn, idiff, idisc, know_full,
      uw, ub, iw, ib, c1w, c1b, c2w, c2b)


def kernel(user_ids, item_ids, item_know, priori, condi_p, condi_n,
           item_diff_w, item_disc_w, user_contract_w, user_contract_b,
           item_contract_w, item_contract_b, cross1_w, cross1_b, cross2_w,
           cross2_b):
    uid = user_ids.astype(jnp.int32)
    iid = item_ids.astype(jnp.int32)
    pri_flat = priori.reshape(-1)
    disc_flat = item_disc_w.reshape(-1)
    ub = user_contract_b.reshape(1, H)
    ib = item_contract_b.reshape(1, H)
    c1b = cross1_b.reshape(1, H // 2)
    c2b = cross2_b.reshape(1, 1)

    outs = []
    for h in range(NCHUNK):
        uid_c = lax.slice(uid, (h * BC,), ((h + 1) * BC,))
        iid_c = lax.slice(iid, (h * BC,), ((h + 1) * BC,))
        idiff, pri0, idisc = _sc_gather(
            uid_c, iid_c, item_diff_w, pri_flat, disc_flat)
        cp = condi_p.at[uid_c].get(mode="promise_in_bounds")
        cn = condi_n.at[uid_c].get(mode="promise_in_bounds")
        outs.append(_tc_forward(
            h, pri0[:, None], cp, cn, idiff, idisc[:, None], item_know,
            user_contract_w, ub, item_contract_w, ib,
            cross1_w, c1b, cross2_w, c2b))
    return jnp.concatenate(outs, axis=0) if NCHUNK > 1 else outs[0]


# speedup vs baseline: 1.0996x; 1.0996x over previous
import functools

import jax
import jax.numpy as jnp
from jax import lax
from jax.experimental import pallas as pl
from jax.experimental.pallas import tpu as pltpu
from jax.experimental.pallas import tpu_sc as plsc

B = 16384
NK = 128
NE = NK - 1
H = 64
BT = 1024
NCHUNK = 2
BC = B // NCHUNK

NW = 32
CH = 128


def _sc_gather(uid, iid, item_diff_w, pri_flat, disc_flat):
    bc = uid.shape[0]
    bpw = bc // NW
    nch = bpw // CH
    mesh = plsc.VectorSubcoreMesh(core_axis_name="c", subcore_axis_name="s")
    f32 = jnp.float32
    out_type = (
        jax.ShapeDtypeStruct((bc, NK), f32),
        jax.ShapeDtypeStruct((bc,), f32),
        jax.ShapeDtypeStruct((bc,), f32),
    )
    scratch = [
        pltpu.VMEM((nch, CH), jnp.int32),
        pltpu.VMEM((nch, CH), jnp.int32),
        pltpu.VMEM((nch, CH), jnp.int32),
        pltpu.VMEM((2, CH, NK), f32),
        pltpu.VMEM((2, CH), f32),
        pltpu.VMEM((2, CH), f32),
    ] + [pltpu.SemaphoreType.DMA] * 4

    @functools.partial(pl.kernel, out_type=out_type, mesh=mesh,
                       scratch_types=scratch)
    def k(uid_h, iid_h, idf_h, pri_h, dsc_h,
          idf_o, pri_o, dsc_o,
          idxu, idxi, pidx, idfb, prib, dscb,
          sg0, sg1, sw0, sw1):
        wid = lax.axis_index("s") * 2 + lax.axis_index("c")
        base = wid * bpw
        for c in range(nch):
            pltpu.sync_copy(uid_h.at[pl.ds(base + c * CH, CH)], idxu.at[c])
            pltpu.sync_copy(iid_h.at[pl.ds(base + c * CH, CH)], idxi.at[c])
        for c in range(nch):
            for j in range(CH // 16):
                sl = pl.ds(j * 16, 16)
                pidx[c, sl] = idxu[c, sl] * NK
        gsem = (sg0, sg1)
        wsem = (sw0, sw1)
        gh = [None, None]
        wh = [None, None]

        def fire_gather(c):
            b = c % 2
            s = gsem[b]
            gh[b] = [
                pltpu.async_copy(idf_h.at[idxi.at[c]], idfb.at[b], s),
                pltpu.async_copy(pri_h.at[pidx.at[c]], prib.at[b], s),
                pltpu.async_copy(dsc_h.at[idxi.at[c]], dscb.at[b], s),
            ]

        def fire_writeback(c):
            b = c % 2
            rb = base + c * CH
            s = wsem[b]
            wh[b] = [
                pltpu.async_copy(idfb.at[b], idf_o.at[pl.ds(rb, CH)], s),
                pltpu.async_copy(prib.at[b], pri_o.at[pl.ds(rb, CH)], s),
                pltpu.async_copy(dscb.at[b], dsc_o.at[pl.ds(rb, CH)], s),
            ]

        def drain(handles):
            for h in handles:
                h.wait()

        fire_gather(0)
        for c in range(nch):
            b = c % 2
            if c + 1 < nch:
                if wh[1 - b] is not None:
                    drain(wh[1 - b])
                fire_gather(c + 1)
            drain(gh[b])
            fire_writeback(c)
        for w in wh:
            if w is not None:
                drain(w)

    return k(uid, iid, item_diff_w, pri_flat, disc_flat)


def _tc_body(pri0_ref, cp_ref, cn_ref, idiff_ref, idisc_ref, know_ref,
             uw_ref, ub_ref, iw_ref, ib_ref, c1w_ref, c1b_ref, c2w_ref,
             c2b_ref, out_ref):
    sig = jax.nn.sigmoid
    bt = pri0_ref.shape[0]
    cp = sig(cp_ref[...])
    cn = sig(cn_ref[...])
    pri0 = sig(pri0_ref[...])
    a = jnp.concatenate([jnp.zeros((bt, 1), jnp.float32), cp - cn], axis=1)
    b = jnp.concatenate([pri0, cn], axis=1)
    d = 1
    for _ in range(7):
        a_sh = jnp.concatenate(
            [jnp.ones((bt, d), jnp.float32), a[:, :NK - d]], axis=1)
        b_sh = jnp.concatenate(
            [jnp.zeros((bt, d), jnp.float32), b[:, :NK - d]], axis=1)
        b = a * b_sh + b
        a = a * a_sh
        d *= 2
    mastery = b

    know = know_ref[...]
    dn = (((1,), (1,)), ((), ()))
    uf = jnp.tanh(
        lax.dot_general(mastery * know, uw_ref[...], dn,
                        preferred_element_type=jnp.float32) + ub_ref[...])
    itf = sig(
        lax.dot_general(sig(idiff_ref[...]) * know, iw_ref[...], dn,
                        preferred_element_type=jnp.float32) + ib_ref[...])
    inp = (uf - itf) * sig(idisc_ref[...])
    x1 = sig(
        lax.dot_general(inp, c1w_ref[...], dn,
                        preferred_element_type=jnp.float32) + c1b_ref[...])
    out_ref[...] = sig(
        jnp.sum(x1 * c2w_ref[...], axis=1, keepdims=True) + c2b_ref[...])


def _tc_forward(chunk, pri0, cp, cn, idiff, idisc, know_full,
                uw, ub, iw, ib, c1w, c1b, c2w, c2b):
    bc = pri0.shape[0]
    nb0 = chunk * (bc // BT)
    row = lambda shape: pl.BlockSpec(shape, lambda i: (i, 0))
    krow = pl.BlockSpec((BT, NK), lambda i: (nb0 + i, 0))
    full = lambda shape: pl.BlockSpec(shape, lambda i: (0, 0))
    in_specs = [
        row((BT, 1)),
        row((BT, NE)),
        row((BT, NE)),
        row((BT, NK)),
        row((BT, 1)),
        krow,
        full((H, NK)),
        full((1, H)),
        full((H, NK)),
        full((1, H)),
        full((H // 2, H)),
        full((1, H // 2)),
        full((1, H // 2)),
        full((1, 1)),
    ]
    return pl.pallas_call(
        _tc_body,
        grid=(bc // BT,),
        in_specs=in_specs,
        out_specs=row((BT, 1)),
        out_shape=jax.ShapeDtypeStruct((bc, 1), jnp.float32),
    )(pri0, cp, cn, idiff, idisc, know_full,
      uw, ub, iw, ib, c1w, c1b, c2w, c2b)


def kernel(user_ids, item_ids, item_know, priori, condi_p, condi_n,
           item_diff_w, item_disc_w, user_contract_w, user_contract_b,
           item_contract_w, item_contract_b, cross1_w, cross1_b, cross2_w,
           cross2_b):
    uid = user_ids.astype(jnp.int32)
    iid = item_ids.astype(jnp.int32)
    pri_flat = priori.reshape(-1)
    disc_flat = item_disc_w.reshape(-1)
    ub = user_contract_b.reshape(1, H)
    ib = item_contract_b.reshape(1, H)
    c1b = cross1_b.reshape(1, H // 2)
    c2b = cross2_b.reshape(1, 1)

    outs = []
    for h in range(NCHUNK):
        uid_c = lax.slice(uid, (h * BC,), ((h + 1) * BC,))
        iid_c = lax.slice(iid, (h * BC,), ((h + 1) * BC,))
        idiff, pri0, idisc = _sc_gather(
            uid_c, iid_c, item_diff_w, pri_flat, disc_flat)
        cp = condi_p.at[uid_c].get(mode="promise_in_bounds")
        cn = condi_n.at[uid_c].get(mode="promise_in_bounds")
        outs.append(_tc_forward(
            h, pri0[:, None], cp, cn, idiff, idisc[:, None], item_know,
            user_contract_w, ub, item_contract_w, ib,
            cross1_w, c1b, cross2_w, c2b))
    return jnp.concatenate(outs, axis=0) if NCHUNK > 1 else outs[0]


# speedup vs baseline: 1.1660x; 1.0604x over previous
import functools

import jax
import jax.numpy as jnp
from jax import lax
from jax.experimental import pallas as pl
from jax.experimental.pallas import tpu as pltpu
from jax.experimental.pallas import tpu_sc as plsc

B = 16384
NK = 128
NE = NK - 1
H = 64
BT = 1024
NCHUNK = 2
BC = B // NCHUNK

NW = 32
CH = 128


def _sc_gather(uid, iid, item_diff_w, pri_flat, disc_flat):
    bc = uid.shape[0]
    bpw = bc // NW
    nch = bpw // CH
    mesh = plsc.VectorSubcoreMesh(core_axis_name="c", subcore_axis_name="s")
    f32 = jnp.float32
    out_type = (
        jax.ShapeDtypeStruct((bc, NK), f32),
        jax.ShapeDtypeStruct((bc,), f32),
        jax.ShapeDtypeStruct((bc,), f32),
    )
    scratch = [
        pltpu.VMEM((nch, CH), jnp.int32),
        pltpu.VMEM((nch, CH), jnp.int32),
        pltpu.VMEM((nch, CH), jnp.int32),
        pltpu.VMEM((2, CH, NK), f32),
        pltpu.VMEM((2, CH), f32),
        pltpu.VMEM((2, CH), f32),
    ] + [pltpu.SemaphoreType.DMA] * 4

    @functools.partial(pl.kernel, out_type=out_type, mesh=mesh,
                       scratch_types=scratch)
    def k(uid_h, iid_h, idf_h, pri_h, dsc_h,
          idf_o, pri_o, dsc_o,
          idxu, idxi, pidx, idfb, prib, dscb,
          sg0, sg1, sw0, sw1):
        wid = lax.axis_index("s") * 2 + lax.axis_index("c")
        base = wid * bpw
        for c in range(nch):
            pltpu.sync_copy(uid_h.at[pl.ds(base + c * CH, CH)], idxu.at[c])
            pltpu.sync_copy(iid_h.at[pl.ds(base + c * CH, CH)], idxi.at[c])
        for c in range(nch):
            for j in range(CH // 16):
                sl = pl.ds(j * 16, 16)
                pidx[c, sl] = idxu[c, sl] * NK
        gsem = (sg0, sg1)
        wsem = (sw0, sw1)
        gh = [None, None]
        wh = [None, None]

        def fire_gather(c):
            b = c % 2
            s = gsem[b]
            gh[b] = [
                pltpu.async_copy(idf_h.at[idxi.at[c]], idfb.at[b], s),
                pltpu.async_copy(pri_h.at[pidx.at[c]], prib.at[b], s),
                pltpu.async_copy(dsc_h.at[idxi.at[c]], dscb.at[b], s),
            ]

        def fire_writeback(c):
            b = c % 2
            rb = base + c * CH
            s = wsem[b]
            wh[b] = [
                pltpu.async_copy(idfb.at[b], idf_o.at[pl.ds(rb, CH)], s),
                pltpu.async_copy(prib.at[b], pri_o.at[pl.ds(rb, CH)], s),
                pltpu.async_copy(dscb.at[b], dsc_o.at[pl.ds(rb, CH)], s),
            ]

        def drain(handles):
            for h in handles:
                h.wait()

        fire_gather(0)
        for c in range(nch):
            b = c % 2
            if c + 1 < nch:
                if wh[1 - b] is not None:
                    drain(wh[1 - b])
                fire_gather(c + 1)
            drain(gh[b])
            fire_writeback(c)
        for w in wh:
            if w is not None:
                drain(w)

    return k(uid, iid, item_diff_w, pri_flat, disc_flat)


def _tc_body(pri0_ref, cp_ref, cn_ref, idiff_ref, idisc_ref, know_ref,
             uw_ref, ub_ref, iw_ref, ib_ref, c1w_ref, c1b_ref, c2w_ref,
             c2b_ref, out_ref):
    sig = jax.nn.sigmoid
    bt = cp_ref.shape[0]
    cp = sig(cp_ref[...])
    cn = sig(cn_ref[...])
    pri0 = sig(pri0_ref[...].reshape(bt, 1))
    a = jnp.concatenate([jnp.zeros((bt, 1), jnp.float32), cp - cn], axis=1)
    b = jnp.concatenate([pri0, cn], axis=1)
    d = 1
    for _ in range(7):
        a_sh = jnp.concatenate(
            [jnp.ones((bt, d), jnp.float32), a[:, :NK - d]], axis=1)
        b_sh = jnp.concatenate(
            [jnp.zeros((bt, d), jnp.float32), b[:, :NK - d]], axis=1)
        b = a * b_sh + b
        a = a * a_sh
        d *= 2
    mastery = b

    know = know_ref[...]
    dn = (((1,), (1,)), ((), ()))
    uf = jnp.tanh(
        lax.dot_general(mastery * know, uw_ref[...], dn,
                        preferred_element_type=jnp.float32) + ub_ref[...])
    itf = sig(
        lax.dot_general(sig(idiff_ref[...]) * know, iw_ref[...], dn,
                        preferred_element_type=jnp.float32) + ib_ref[...])
    inp = (uf - itf) * sig(idisc_ref[...].reshape(bt, 1))
    x1 = sig(
        lax.dot_general(inp, c1w_ref[...], dn,
                        preferred_element_type=jnp.float32) + c1b_ref[...])
    out_ref[...] = sig(
        jnp.sum(x1 * c2w_ref[...], axis=1) + c2b_ref[...].reshape(1))


def _tc_forward(chunk, pri0, cp, cn, idiff, idisc, know_full,
                uw, ub, iw, ib, c1w, c1b, c2w, c2b):
    bc = pri0.shape[0]
    nb0 = chunk * (bc // BT)
    row = lambda shape: pl.BlockSpec(shape, lambda i: (i, 0))
    vec = pl.BlockSpec((BT,), lambda i: (i,))
    krow = pl.BlockSpec((BT, NK), lambda i: (nb0 + i, 0))
    full = lambda shape: pl.BlockSpec(shape, lambda i: (0, 0))
    in_specs = [
        vec,
        row((BT, NE)),
        row((BT, NE)),
        row((BT, NK)),
        vec,
        krow,
        full((H, NK)),
        full((1, H)),
        full((H, NK)),
        full((1, H)),
        full((H // 2, H)),
        full((1, H // 2)),
        full((1, H // 2)),
        full((1, 1)),
    ]
    return pl.pallas_call(
        _tc_body,
        grid=(bc // BT,),
        in_specs=in_specs,
        out_specs=vec,
        out_shape=jax.ShapeDtypeStruct((bc,), jnp.float32),
    )(pri0, cp, cn, idiff, idisc, know_full,
      uw, ub, iw, ib, c1w, c1b, c2w, c2b)


def kernel(user_ids, item_ids, item_know, priori, condi_p, condi_n,
           item_diff_w, item_disc_w, user_contract_w, user_contract_b,
           item_contract_w, item_contract_b, cross1_w, cross1_b, cross2_w,
           cross2_b):
    uid = user_ids.astype(jnp.int32)
    iid = item_ids.astype(jnp.int32)
    pri_flat = priori.reshape(-1)
    disc_flat = item_disc_w.reshape(-1)
    ub = user_contract_b.reshape(1, H)
    ib = item_contract_b.reshape(1, H)
    c1b = cross1_b.reshape(1, H // 2)
    c2b = cross2_b.reshape(1, 1)

    outs = []
    for h in range(NCHUNK):
        uid_c = lax.slice(uid, (h * BC,), ((h + 1) * BC,))
        iid_c = lax.slice(iid, (h * BC,), ((h + 1) * BC,))
        idiff, pri0, idisc = _sc_gather(
            uid_c, iid_c, item_diff_w, pri_flat, disc_flat)
        cp = condi_p.at[uid_c].get(mode="promise_in_bounds")
        cn = condi_n.at[uid_c].get(mode="promise_in_bounds")
        outs.append(_tc_forward(
            h, pri0, cp, cn, idiff, idisc, item_know,
            user_contract_w, ub, item_contract_w, ib,
            cross1_w, c1b, cross2_w, c2b))
    out = jnp.concatenate(outs, axis=0) if NCHUNK > 1 else outs[0]
    return out[:, None]
